# baseline (speedup 1.0000x reference)
import functools

import jax
import jax.numpy as jnp
from jax import lax
from jax.experimental import pallas as pl
from jax.experimental.pallas import tpu as pltpu
from jax.experimental.pallas import tpu_sc as plsc

B = 4096
N = 20
D = 64
NC = 2
NS = 16
L = 16
NW = NC * NS
BPW = B // NW
CB = 32
NCHUNK = BPW // CB
ROWS = CB * N
NGRP = ROWS // 128
NHALF = N // 2


def _body(ctx_hbm, ids_hbm, table_hbm, out_hbm,
          idx_v, rows_v, ctx_v, scores_v, out_v, sem):
    cid = lax.axis_index("c")
    sid = lax.axis_index("s")
    wid = sid * NC + cid
    b0 = wid * BPW

    pltpu.sync_copy(ctx_hbm.at[pl.ds(b0, BPW), :], ctx_v)

    lane = lax.iota(jnp.int32, L)

    for chunk in range(NCHUNK):
        cb0 = b0 + chunk * CB
        pltpu.sync_copy(ids_hbm.at[pl.ds(cb0 * N, ROWS)], idx_v)
        copies = []
        for g in range(NGRP):
            cp = pltpu.make_async_copy(
                table_hbm.at[idx_v.at[pl.ds(g * 128, 128)]],
                rows_v.at[pl.ds(g * 128, 128), :],
                sem,
            )
            cp.start()
            copies.append(cp)
        for cp in copies:
            cp.wait()

        for bb in range(CB // L):
            bcol = lane + (chunk * CB + bb * L)
            rbase = (lane + bb * L) * N

            for h in range(N // NHALF):
                def dbody(d, accs, _h=h, _bcol=bcol, _rbase=rbase):
                    dcol = jnp.full((L,), 0, jnp.int32) + d
                    cvec = plsc.load_gather(ctx_v, [_bcol, dcol])
                    return tuple(
                        accs[i]
                        + plsc.load_gather(
                            rows_v, [_rbase + (_h * NHALF + i), dcol]) * cvec
                        for i in range(NHALF)
                    )

                accs = lax.fori_loop(
                    0, D, dbody,
                    tuple(jnp.zeros((L,), jnp.float32) for _ in range(NHALF)))
                for i in range(NHALF):
                    scores_v[h * NHALF + i, :] = accs[i]

            m = scores_v[0, :]
            for n in range(1, N):
                m = jnp.maximum(m, scores_v[n, :])
            tot = jnp.zeros((L,), jnp.float32)
            es = []
            for n in range(N):
                e = jnp.exp(scores_v[n, :] - m)
                es.append(e)
                tot = tot + e
            r = 1.0 / tot
            blocal = lane + bb * L
            for n in range(N):
                ncol = jnp.full((L,), n, jnp.int32)
                plsc.store_scatter(out_v, [blocal, ncol], es[n] * r)

        pltpu.sync_copy(out_v, out_hbm.at[pl.ds(cb0, CB), :])


@jax.jit
def _entity_posterior_sc(context_encoded, ids_flat, entity_embeddings):
    mesh = plsc.VectorSubcoreMesh(
        core_axis_name="c", subcore_axis_name="s",
        num_cores=NC, num_subcores=NS)
    return pl.kernel(
        _body,
        out_type=jax.ShapeDtypeStruct((B, N), jnp.float32),
        mesh=mesh,
        scratch_types=[
            pltpu.VMEM((ROWS,), jnp.int32),
            pltpu.VMEM((ROWS, D), jnp.float32),
            pltpu.VMEM((BPW, D), jnp.float32),
            pltpu.VMEM((N, L), jnp.float32),
            pltpu.VMEM((CB, N), jnp.float32),
            pltpu.SemaphoreType.DMA,
        ],
        compiler_params=pltpu.CompilerParams(
            needs_layout_passes=False, use_tc_tiling_on_sc=False),
        name="entity_posterior_sc",
    )(context_encoded, ids_flat, entity_embeddings)


def kernel(context_encoded, entity_ids, entity_embeddings):
    ids_flat = entity_ids.reshape(-1)
    return _entity_posterior_sc(context_encoded, ids_flat, entity_embeddings)
